# baseline (speedup 1.0000x reference)
import functools

import jax
import jax.numpy as jnp
from jax import lax
from jax.experimental import pallas as pl
from jax.experimental.pallas import tpu as pltpu
from jax.experimental.pallas import tpu_sc as plsc

_B, _L, _V = 16384, 77, 1000
_VP = 1024
_D, _DFF = 256, 512

_NC, _NS = 2, 16
_NW = _NC * _NS
_RPW = _B // _NW
_R = 32
_NG = _RPW // _R
_CST = (0, 16, 32, 48, 61)


_TCH = 64


def _hist_body(tokens_hbm, counts_hbm,
               tok_v0, tok_v1, tokt_v, cnt_v0, cnt_v1,
               tsem0, tsem1, sem0, sem1):
    wid = lax.axis_index("s") * _NC + lax.axis_index("c")
    iota = lax.iota(jnp.int32, 16)
    ones = jnp.ones((16,), jnp.float32)
    zeros = jnp.zeros((16,), jnp.float32)
    row_base = wid * _RPW

    toks = (tok_v0, tok_v1)
    tsems = (tsem0, tsem1)
    tcopies = [
        pltpu.async_copy(tokens_hbm.at[pl.ds(row_base, _TCH)], tok_v0, tsem0),
        None,
    ]
    for ch in range(_RPW // _TCH):
        bsel = ch % 2
        tcopies[bsel].wait()
        if ch + 1 < _RPW // _TCH:
            nb = (ch + 1) % 2
            tcopies[nb] = pltpu.async_copy(
                tokens_hbm.at[pl.ds(row_base + (ch + 1) * _TCH, _TCH)],
                toks[nb], tsems[nb])

        def tbody(r, c, tok_v=toks[bsel], ch=ch):
            gr = ch * _TCH + r
            for st in _CST:
                v = tok_v[r, pl.ds(st, 16)]
                plsc.store_scatter(tokt_v, [(st + iota) * _RPW + gr], v)
            return c

        lax.fori_loop(0, _TCH, tbody, 0, unroll=2)

    for cnt_v in (cnt_v0, cnt_v1):
        def zbody(r, c, cnt_v=cnt_v):
            for k in range(_VP // 16):
                cnt_v[r, pl.ds(k * 16, 16)] = zeros
            return c

        lax.fori_loop(0, _R, zbody, 0)

    cnts = (cnt_v0, cnt_v1)
    csems = (sem0, sem1)
    ccopies = [None, None]
    for g in range(_NG):
        bsel = g % 2
        cnt_v = cnts[bsel]
        if ccopies[bsel] is not None:
            ccopies[bsel].wait()
            for rb in range(_R // 16):
                def zsbody(l, c, cnt_v=cnt_v, rb=rb, gp=g - 2):
                    tok = tokt_v[pl.ds(l * _RPW + gp * _R + rb * 16, 16)]
                    plsc.store_scatter(cnt_v, [rb * 16 + iota, tok], zeros)
                    return c

                lax.fori_loop(0, _L, zsbody, 0, unroll=7)
        for rb in range(_R // 16):
            def sbody(l, c, cnt_v=cnt_v, rb=rb, g=g):
                tok = tokt_v[pl.ds(l * _RPW + g * _R + rb * 16, 16)]
                plsc.addupdate_scatter(cnt_v, [rb * 16 + iota, tok], ones)
                return c

            lax.fori_loop(0, _L, sbody, 0, unroll=7)
        ccopies[bsel] = pltpu.async_copy(
            cnt_v, counts_hbm.at[pl.ds(row_base + g * _R, _R)], csems[bsel])
    ccopies[0].wait()
    ccopies[1].wait()


@functools.lru_cache(maxsize=None)
def _get_hist():
    return functools.partial(
        pl.kernel,
        mesh=plsc.VectorSubcoreMesh(core_axis_name="c", subcore_axis_name="s"),
        out_type=jax.ShapeDtypeStruct((_B, _VP), jnp.float32),
        scratch_types=[
            pltpu.VMEM((_TCH, _L), jnp.int32),
            pltpu.VMEM((_TCH, _L), jnp.int32),
            pltpu.VMEM((_L * _RPW,), jnp.int32),
            pltpu.VMEM((_R, _VP), jnp.float32),
            pltpu.VMEM((_R, _VP), jnp.float32),
            pltpu.SemaphoreType.DMA,
            pltpu.SemaphoreType.DMA,
            pltpu.SemaphoreType.DMA,
            pltpu.SemaphoreType.DMA,
        ],
        compiler_params=pltpu.CompilerParams(needs_layout_passes=False),
    )(_hist_body)


def _t4_body(table_ref, w1_ref, b1_ref, w2_ref, o_ref):
    acc = jnp.dot(table_ref[...], w1_ref[...], preferred_element_type=jnp.float32)
    h = jnp.maximum(acc + b1_ref[...], 0.0) * (1.0 / _L)
    t4 = jnp.dot(h, w2_ref[...], preferred_element_type=jnp.float32)
    o_ref[...] = t4.astype(jnp.bfloat16)


_t4 = pl.pallas_call(
    _t4_body,
    out_shape=jax.ShapeDtypeStruct((_VP, _D), jnp.bfloat16),
)

_BM = 1024


def _mlp_body(cnt_ref, t4_ref, b2_ref, o_ref):
    cnt_bf = cnt_ref[...].astype(jnp.bfloat16)
    o_ref[...] = (
        jnp.dot(cnt_bf, t4_ref[...], preferred_element_type=jnp.float32)
        + b2_ref[...]
    )


_mlp = pl.pallas_call(
    _mlp_body,
    grid=(_B // _BM,),
    in_specs=[
        pl.BlockSpec((_BM, _VP), lambda i: (i, 0)),
        pl.BlockSpec((_VP, _D), lambda i: (0, 0)),
        pl.BlockSpec((1, _D), lambda i: (0, 0)),
    ],
    out_specs=pl.BlockSpec((_BM, _D), lambda i: (i, 0)),
    out_shape=jax.ShapeDtypeStruct((_B, _D), jnp.float32),
)


def kernel(tokens, table, W1, b1, W2, b2):
    table_p = jnp.pad(table, ((0, _VP - _V), (0, 0)))
    table4 = _t4(table_p, W1, b1.reshape(1, -1), W2)
    counts = _get_hist()(tokens)
    return _mlp(counts, table4, b2.reshape(1, -1))
